# baseline (speedup 1.0000x reference)
import functools

import jax
import jax.numpy as jnp
from jax import lax
from jax.experimental import pallas as pl
from jax.experimental.pallas import tpu as pltpu
from jax.experimental.pallas import tpu_sc as plsc

_LANES = 16


def kernel(x, token_table, pos_table):
    B, T = x.shape
    V, D = token_table.shape
    assert pos_table.shape == (T, D)
    x_flat = x.astype(jnp.int32).reshape(B * T)

    info = plsc.get_sparse_core_info()
    NC, NS = info.num_cores, info.num_subcores
    NW = NC * NS
    rows_per_w = B // NW
    G0 = 128
    G1 = T - G0

    mesh = plsc.VectorSubcoreMesh(core_axis_name="core", subcore_axis_name="subcore")

    @functools.partial(
        pl.kernel,
        out_type=jax.ShapeDtypeStruct((B * T, D), jnp.float32),
        mesh=mesh,
        compiler_params=pltpu.CompilerParams(use_tc_tiling_on_sc=False),
        scratch_types=[
            pltpu.VMEM((T,), jnp.int32),
            pltpu.VMEM((T, D), jnp.float32),
            pltpu.VMEM((T, D), jnp.float32),
            pltpu.SemaphoreType.DMA,
        ],
    )
    def run(tok_hbm, idx_hbm, pos_hbm, out_hbm, idx_v, rows_v, pos_v, sem):
        wid = lax.axis_index("subcore") * NC + lax.axis_index("core")
        base_row = wid * rows_per_w

        pltpu.async_copy(pos_hbm, pos_v, sem).wait()

        @pl.loop(0, rows_per_w)
        def _row(j):
            r = base_row + j
            pltpu.sync_copy(idx_hbm.at[pl.ds(r * T, T)], idx_v)
            pltpu.async_copy(tok_hbm.at[idx_v.at[pl.ds(0, G0)]],
                             rows_v.at[pl.ds(0, G0)], sem).wait()
            pltpu.async_copy(tok_hbm.at[idx_v.at[pl.ds(G0, G1)]],
                             rows_v.at[pl.ds(G0, G1)], sem).wait()

            @pl.loop(0, T)
            def _pos(t):
                for c in range(D // _LANES):
                    sl = pl.ds(c * _LANES, _LANES)
                    rows_v[t, sl] = rows_v[t, sl] + pos_v[t, sl]

            pltpu.sync_copy(rows_v, out_hbm.at[pl.ds(r * T, T)])

    out = run(token_table, x_flat, pos_table)
    return out.reshape(B, T, D)
